# baseline (speedup 1.0000x reference)
import functools

import jax
import jax.numpy as jnp
from jax import lax
from jax.experimental import pallas as pl
from jax.experimental.pallas import tpu as pltpu
from jax.experimental.pallas import tpu_sc as plsc

N_NODES = 10000
N_EDGES = 320000
D = 128
DH = D // 2

NC = 2
NS = 16

K = 64
NBK = 10
NCHT = 512
CPT0 = 24
CPT1 = 8
EB = CPT0 * NBK
E_PAD = NCHT * NBK * K
PAD_COL = N_NODES + 200

N_PAD = 10240
DEG_SL = N_PAD // NS
DGB = NCHT * NBK // NS // 2


def _rsqrt16(d):
    i = lax.bitcast_convert_type(d, jnp.int32)
    magic = jnp.full((16,), 0x5F3759DF, jnp.int32)
    y = lax.bitcast_convert_type(magic - lax.shift_right_logical(i, 1), jnp.float32)
    for _ in range(3):
        y = y * (1.5 - 0.5 * d * y * y)
    return jnp.where(d > 0.5, y, 0.0)


def _sc_body(h0_hbm, h1_hbm, row_hbm, col_hbm, cns_hbm, out_hbm,
             rowB, colB, cnsB, disv, ov,
             gbufa, gbufb, gbufc, gbufd, sbufa, sbufb, sbufc, sbufd,
             deg_sh, out_sh, esem,
             gsem0, gsem1, gsem2, gsem3, ssem0, ssem1, ssem2, ssem3):
    c = lax.axis_index("c")
    s = lax.axis_index("s")
    ebase = jnp.where(c == 0, CPT0 * s, NS * CPT0 + CPT1 * s) * NBK
    cntb = jnp.where(c == 0, CPT0 * NBK, CPT1 * NBK)
    gbufs = (gbufa, gbufb, gbufc, gbufd)
    sbufs = (sbufa, sbufb, sbufc, sbufd)
    gsems = (gsem0, gsem1, gsem2, gsem3)
    ssems = (ssem0, ssem1, ssem2, ssem3)

    one16 = jnp.ones((16,), jnp.float32)
    for j in range(K // 16):
        ov[pl.ds(j * 16, 16)] = one16

    def _zv_zero(i, _):
        disv[pl.ds(i * 16, 16)] = jnp.zeros((16,), jnp.float32)
        return _
    lax.fori_loop(0, DEG_SL // 16, _zv_zero, None)

    def _gbufa_zero(e, _):
        for j in range(DH // 16):
            gbufa[e, pl.ds(j * 16, 16)] = jnp.zeros((16,), jnp.float32)
        return _
    lax.fori_loop(0, K, _gbufa_zero, None)

    obase = s * DEG_SL

    def _zero_out_sh():
        def _fire(q, _):
            pltpu.async_copy(gbufa, out_sh.at[pl.ds(obase + q * K, K)], esem)
            return _
        lax.fori_loop(0, DEG_SL // K, _fire, None)

        def _drain(q, _):
            pltpu.make_async_copy(gbufa, out_sh.at[pl.ds(obase, K)], esem).wait()
            return _
        lax.fori_loop(0, DEG_SL // K, _drain, None)

    with jax.named_scope("zero"):
        pltpu.sync_copy(disv.at[pl.ds(0, DEG_SL)],
                        deg_sh.at[pl.ds(s * DEG_SL, DEG_SL)])
        _zero_out_sh()
        plsc.subcore_barrier()

    with jax.named_scope("deg"):
        pltpu.sync_copy(col_hbm.at[pl.ds(2 * DGB * s, DGB)], rowB.at[pl.ds(0, DGB)])
        pltpu.sync_copy(col_hbm.at[pl.ds(2 * DGB * s + DGB, DGB)],
                        colB.at[pl.ds(0, DGB)])

        def _deg_fire(i, _):
            pltpu.async_copy(ov, deg_sh.at[rowB.at[i]], esem, add=True)
            pltpu.async_copy(ov, deg_sh.at[colB.at[i]], esem, add=True)
            return _
        lax.fori_loop(0, DGB, _deg_fire, None)

        def _deg_drain(i, _):
            pltpu.make_async_copy(ov, deg_sh.at[rowB.at[0]], esem).wait()
            return _
        lax.fori_loop(0, 2 * DGB, _deg_drain, None)
    plsc.subcore_barrier()

    with jax.named_scope("dis"):
        doff = s * DEG_SL
        pltpu.sync_copy(deg_sh.at[pl.ds(doff, DEG_SL)], disv.at[pl.ds(0, DEG_SL)])

        def _dis(i, _):
            sl = pl.ds(i * 16, 16)
            disv[sl] = _rsqrt16(disv[sl])
            return _
        lax.fori_loop(0, DEG_SL // 16, _dis, None)
        pltpu.sync_copy(disv.at[pl.ds(0, DEG_SL)], deg_sh.at[pl.ds(doff, DEG_SL)])
        plsc.subcore_barrier()
        pltpu.sync_copy(deg_sh, disv)

    with jax.named_scope("norm"):
        @pl.when(c == 0)
        def _():
            pltpu.sync_copy(row_hbm.at[pl.ds(ebase, CPT0 * NBK)], rowB)
            pltpu.sync_copy(col_hbm.at[pl.ds(ebase, CPT0 * NBK)], colB)
            pltpu.sync_copy(cns_hbm.at[pl.ds(ebase, CPT0 * NBK)], cnsB)

        @pl.when(c == 1)
        def _():
            pltpu.sync_copy(row_hbm.at[pl.ds(ebase, CPT1 * NBK)],
                            rowB.at[pl.ds(0, CPT1 * NBK)])
            pltpu.sync_copy(col_hbm.at[pl.ds(ebase, CPT1 * NBK)],
                            colB.at[pl.ds(0, CPT1 * NBK)])
            pltpu.sync_copy(cns_hbm.at[pl.ds(ebase, CPT1 * NBK)],
                            cnsB.at[pl.ds(0, CPT1 * NBK)])

        def _norm(i, _):
            for g in range(K // 16):
                sl = pl.ds(g * 16, 16)
                dr = plsc.load_gather(disv, [rowB[i, sl]])
                dc = plsc.load_gather(disv, [colB[i, sl]])
                cnsB[i, sl] = dr * dc * jnp.exp(cnsB[i, sl])
            return _
        lax.fori_loop(0, cntb, _norm, None)

    for hf, href in enumerate((h0_hbm, h1_hbm)):
        with jax.named_scope(f"half{hf}"):
            def _sbuf_zero(e, _):
                for j in range(DH // 16):
                    z = jnp.zeros((16,), jnp.float32)
                    sbufa[e, pl.ds(j * 16, 16)] = z
                    sbufb[e, pl.ds(j * 16, 16)] = z
                    sbufc[e, pl.ds(j * 16, 16)] = z
                    sbufd[e, pl.ds(j * 16, 16)] = z
                return _
            lax.fori_loop(0, K, _sbuf_zero, None)
            for t in range(4):
                pltpu.async_copy(sbufs[t], out_sh.at[colB.at[0]], ssems[t],
                                 add=True)
                pltpu.async_copy(href.at[rowB.at[t]], gbufs[t], gsems[t])

            def _quad(q, _):
                for t in range(4):
                    b = 4 * q + t
                    pltpu.make_async_copy(href.at[rowB.at[0]], gbufs[t],
                                          gsems[t]).wait()
                    pltpu.make_async_copy(sbufs[t], out_sh.at[colB.at[0]],
                                          ssems[t]).wait()
                    gbuf, sbuf = gbufs[t], sbufs[t]
                    for g in range(K // 16):
                        nv = cnsB[b, pl.ds(g * 16, 16)]
                        r0 = g * 16
                        for e in range(16):
                            sc = nv[e]
                            for jj in range(DH // 16):
                                sl = pl.ds(jj * 16, 16)
                                sbuf[r0 + e, sl] = gbuf[r0 + e, sl] * sc
                    pltpu.async_copy(sbufs[t], out_sh.at[colB.at[b]], ssems[t],
                                     add=True)

                    @pl.when(b + 4 < cntb)
                    def _():
                        pltpu.async_copy(href.at[rowB.at[b + 4]], gbufs[t],
                                         gsems[t])
                return _
            lax.fori_loop(0, cntb // 4, _quad, None)

            for t in range(4):
                pltpu.make_async_copy(sbufs[t], out_sh.at[colB.at[0]],
                                      ssems[t]).wait()
            plsc.subcore_barrier()

        with jax.named_scope(f"dump{hf}"):
            pltpu.sync_copy(out_sh.at[pl.ds(obase, DEG_SL)],
                            out_hbm.at[pl.ds((c * 2 + hf) * N_PAD + obase, DEG_SL)])
        if hf == 0:
            def _gbufa_rezero(e, _):
                for j in range(DH // 16):
                    gbufa[e, pl.ds(j * 16, 16)] = jnp.zeros((16,), jnp.float32)
                return _
            lax.fori_loop(0, K, _gbufa_rezero, None)
            _zero_out_sh()
            plsc.subcore_barrier()


_sc_propagate = functools.partial(
    pl.kernel,
    out_type=jax.ShapeDtypeStruct((NC * 2 * N_PAD, DH), jnp.float32),
    mesh=plsc.VectorSubcoreMesh(core_axis_name="c", subcore_axis_name="s"),
    compiler_params=pltpu.CompilerParams(needs_layout_passes=False,
                                         use_tc_tiling_on_sc=False),
    scratch_types=[
        pltpu.VMEM((EB, K), jnp.int32),
        pltpu.VMEM((EB, K), jnp.int32),
        pltpu.VMEM((EB, K), jnp.float32),
        pltpu.VMEM((N_PAD,), jnp.float32),
        pltpu.VMEM((K,), jnp.float32),
        pltpu.VMEM((K, DH), jnp.float32),
        pltpu.VMEM((K, DH), jnp.float32),
        pltpu.VMEM((K, DH), jnp.float32),
        pltpu.VMEM((K, DH), jnp.float32),
        pltpu.VMEM((K, DH), jnp.float32),
        pltpu.VMEM((K, DH), jnp.float32),
        pltpu.VMEM((K, DH), jnp.float32),
        pltpu.VMEM((K, DH), jnp.float32),
        pltpu.VMEM_SHARED((N_PAD,), jnp.float32),
        pltpu.VMEM_SHARED((N_PAD, DH), jnp.float32),
        pltpu.SemaphoreType.DMA,
        pltpu.SemaphoreType.DMA,
        pltpu.SemaphoreType.DMA,
        pltpu.SemaphoreType.DMA,
        pltpu.SemaphoreType.DMA,
        pltpu.SemaphoreType.DMA,
        pltpu.SemaphoreType.DMA,
        pltpu.SemaphoreType.DMA,
        pltpu.SemaphoreType.DMA,
    ],
)(_sc_body)


def _mm_body(x_ref, w_ref, o_ref):
    o_ref[...] = lax.dot_general(
        x_ref[...], w_ref[...], (((1,), (1,)), ((), ())),
        preferred_element_type=jnp.float32)


def _comb_body(p00, p01, p10, p11, b_ref, o_ref):
    o_ref[:, :DH] = p00[0, 0] + p10[0, 0] + b_ref[0, :DH]
    o_ref[:, DH:] = p01[0, 0] + p11[0, 0] + b_ref[0, DH:]


def kernel(x, edge_index, cns, W, bias):
    n, d_in = x.shape
    d_out = W.shape[0]
    nblk = 10
    h = pl.pallas_call(
        _mm_body,
        grid=(nblk,),
        in_specs=[
            pl.BlockSpec((n // nblk, d_in), lambda i: (i, 0)),
            pl.BlockSpec((d_out, d_in), lambda i: (0, 0)),
        ],
        out_specs=pl.BlockSpec((n // nblk, d_out), lambda i: (i, 0)),
        out_shape=jax.ShapeDtypeStruct((n, d_out), jnp.float32),
    )(x, W)

    n_edges = edge_index.shape[1]
    pad = E_PAD - n_edges
    row2 = jnp.concatenate(
        [edge_index[0], jnp.zeros((pad,), jnp.int32)]).reshape(NCHT * NBK, K)
    col2 = jnp.concatenate(
        [edge_index[1], jnp.full((pad,), PAD_COL, jnp.int32)]).reshape(NCHT * NBK, K)
    cns2 = jnp.concatenate(
        [cns, jnp.full((pad,), -1e4, cns.dtype)]).reshape(NCHT * NBK, K)
    h0 = h[:, :DH]
    h1 = h[:, DH:]
    part = _sc_propagate(h0, h1, row2, col2, cns2).reshape(NC, 2, N_PAD, DH)

    out = pl.pallas_call(
        _comb_body,
        grid=(nblk,),
        in_specs=[
            pl.BlockSpec((1, 1, n // nblk, DH), lambda i: (0, 0, i, 0)),
            pl.BlockSpec((1, 1, n // nblk, DH), lambda i: (0, 1, i, 0)),
            pl.BlockSpec((1, 1, n // nblk, DH), lambda i: (1, 0, i, 0)),
            pl.BlockSpec((1, 1, n // nblk, DH), lambda i: (1, 1, i, 0)),
            pl.BlockSpec((1, d_out), lambda i: (0, 0)),
        ],
        out_specs=pl.BlockSpec((n // nblk, d_out), lambda i: (i, 0)),
        out_shape=jax.ShapeDtypeStruct((n, d_out), jnp.float32),
    )(part, part, part, part, bias.reshape(1, d_out))
    return out
